# baseline (speedup 1.0000x reference)
import functools

import jax
import jax.numpy as jnp
from jax import lax
from jax.experimental import pallas as pl
from jax.experimental.pallas import tpu as pltpu
from jax.experimental.pallas import tpu_sc as plsc

N_NODES = 10000
N_EDGES = 320000
D = 128

NW = 32
CHUNK = 128
NCHUNK = 79
NCHUNK_P = 81
EPT = 10000
N_PAD = 10112
ROWS_PER_TILE = N_PAD // 16
DEG_N = 10112
ZBUF = 640
PAD_DST = 10000


def _sc_agg_kernel(x_hbm, comb_hbm,
                   agg_out, deg_out,
                   comb_a, comb_b, src_a, dst_a, doff_a, src_b, dst_b,
                   doff_b, rows_a, rows_b, ones_v, zbuf,
                   agg_s, deg_s, sem_a, sem_b, sem_ca, sem_cb):
    c = lax.axis_index("c")
    s = lax.axis_index("s")
    wid = c * 16 + s
    soff = s * DEG_N
    cbase = wid * (NCHUNK_P * CHUNK)

    def idx_copy(j, comb_v, sem):
        pltpu.async_copy(comb_hbm.at[pl.ds(cbase + j * CHUNK, CHUNK)],
                         comb_v, sem)

    def idx_drain(j, comb_v, sem):
        pltpu.make_async_copy(comb_hbm.at[pl.ds(cbase + j * CHUNK, CHUNK)],
                              comb_v, sem).wait()

    def unpack(comb_v, src_c, dst_c, doff_c):
        for t in range(CHUNK // 16):
            cv = comb_v[pl.ds(t * 16, 16)]
            sv = jnp.bitwise_and(cv, 0xFFFF)
            dv = lax.shift_right_logical(cv, 16)
            src_c[pl.ds(t * 16, 16)] = sv
            dst_c[pl.ds(t * 16, 16)] = dv
            doff_c[pl.ds(t * 16, 16)] = dv + soff

    def gather(src_c, rows_v, sem):
        pltpu.async_copy(x_hbm.at[src_c], rows_v, sem)

    def drain(src_c, rows_v, sem):
        pltpu.make_async_copy(x_hbm.at[src_c], rows_v, sem).wait()

    def scatter_rows(dst_c, rows_v):
        pltpu.sync_copy(rows_v, agg_s.at[dst_c], add=True)

    def deg_add(doff_c):
        pltpu.sync_copy(ones_v, deg_s.at[doff_c], add=True)

    idx_copy(0, comb_a, sem_ca)
    idx_copy(1, comb_b, sem_cb)

    ones = jnp.full((16,), 1.0, jnp.float32)
    zeros = jnp.zeros((16,), jnp.float32)
    for t in range(CHUNK // 16):
        ones_v[pl.ds(t * 16, 16)] = ones

    def zrows_body(i, carry):
        r = i // (D // 16)
        t = i % (D // 16)
        rows_b[r, pl.ds(t * 16, 16)] = zeros
        return carry
    lax.fori_loop(0, CHUNK * (D // 16), zrows_body, 0)
    base = s * ROWS_PER_TILE
    for b in range(ROWS_PER_TILE // CHUNK):
        pltpu.sync_copy(rows_b, agg_s.at[pl.ds(base + b * CHUNK, CHUNK)])
    rem = ROWS_PER_TILE % CHUNK
    if rem:
        pltpu.sync_copy(
            rows_b.at[pl.ds(0, rem)],
            agg_s.at[pl.ds(base + (ROWS_PER_TILE // CHUNK) * CHUNK, rem)])

    def zbuf_body(t, carry):
        zbuf[pl.ds(t * 16, 16)] = zeros
        return carry
    lax.fori_loop(0, ZBUF // 16, zbuf_body, 0)
    for i in range(DEG_N // ZBUF):
        pltpu.sync_copy(zbuf, deg_s.at[pl.ds(soff + i * ZBUF, ZBUF)])
    drem = DEG_N % ZBUF
    if drem:
        pltpu.sync_copy(zbuf.at[pl.ds(0, drem)],
                        deg_s.at[pl.ds(soff + (DEG_N // ZBUF) * ZBUF, drem)])

    idx_drain(0, comb_a, sem_ca)
    unpack(comb_a, src_a, dst_a, doff_a)
    gather(src_a, rows_a, sem_a)
    deg_add(doff_a)
    idx_copy(2, comb_a, sem_ca)
    plsc.subcore_barrier()

    def pair_body(k, carry):
        j1 = 2 * k + 1
        idx_drain(j1, comb_b, sem_cb)
        unpack(comb_b, src_b, dst_b, doff_b)
        gather(src_b, rows_b, sem_b)
        deg_add(doff_b)
        idx_copy(j1 + 2, comb_b, sem_cb)
        drain(src_a, rows_a, sem_a)
        scatter_rows(dst_a, rows_a)
        j2 = 2 * k + 2
        idx_drain(j2, comb_a, sem_ca)
        unpack(comb_a, src_a, dst_a, doff_a)
        gather(src_a, rows_a, sem_a)
        deg_add(doff_a)
        idx_copy(j2 + 2, comb_a, sem_ca)
        drain(src_b, rows_b, sem_b)
        scatter_rows(dst_b, rows_b)
        return carry

    lax.fori_loop(0, (NCHUNK - 1) // 2, pair_body, 0)
    drain(src_a, rows_a, sem_a)
    scatter_rows(dst_a, rows_a)
    idx_drain(NCHUNK + 1, comb_a, sem_ca)
    idx_drain(NCHUNK, comb_b, sem_cb)
    plsc.subcore_barrier()

    pltpu.sync_copy(agg_s.at[pl.ds(base, ROWS_PER_TILE)],
                    agg_out.at[c, pl.ds(base, ROWS_PER_TILE)])
    for i in range(DEG_N // ZBUF):
        pltpu.sync_copy(deg_s.at[pl.ds(soff + i * ZBUF, ZBUF)], zbuf)
        pltpu.sync_copy(zbuf, deg_out.at[pl.ds(wid * DEG_N + i * ZBUF, ZBUF)])
    if DEG_N % ZBUF:
        i = DEG_N // ZBUF
        drem = DEG_N % ZBUF
        pltpu.sync_copy(deg_s.at[pl.ds(soff + i * ZBUF, drem)],
                        zbuf.at[pl.ds(0, drem)])
        pltpu.sync_copy(zbuf.at[pl.ds(0, drem)],
                        deg_out.at[pl.ds(wid * DEG_N + i * ZBUF, drem)])


def _sc_agg(x, comb_r):
    mesh = plsc.VectorSubcoreMesh(core_axis_name="c", subcore_axis_name="s")
    fn = functools.partial(
        pl.kernel,
        mesh=mesh,
        out_type=[
            jax.ShapeDtypeStruct((2, N_PAD, D), jnp.float32),
            jax.ShapeDtypeStruct((NW * DEG_N,), jnp.float32),
        ],
        scratch_types=[
            pltpu.VMEM((CHUNK,), jnp.int32),
            pltpu.VMEM((CHUNK,), jnp.int32),
            pltpu.VMEM((CHUNK,), jnp.int32),
            pltpu.VMEM((CHUNK,), jnp.int32),
            pltpu.VMEM((CHUNK,), jnp.int32),
            pltpu.VMEM((CHUNK,), jnp.int32),
            pltpu.VMEM((CHUNK,), jnp.int32),
            pltpu.VMEM((CHUNK,), jnp.int32),
            pltpu.VMEM((CHUNK, D), jnp.float32),
            pltpu.VMEM((CHUNK, D), jnp.float32),
            pltpu.VMEM((CHUNK,), jnp.float32),
            pltpu.VMEM((ZBUF,), jnp.float32),
            pltpu.VMEM_SHARED((N_PAD, D), jnp.float32),
            pltpu.VMEM_SHARED((16 * DEG_N,), jnp.float32),
            pltpu.SemaphoreType.DMA,
            pltpu.SemaphoreType.DMA,
            pltpu.SemaphoreType.DMA,
            pltpu.SemaphoreType.DMA,
        ],
    )
    return fn(_sc_agg_kernel)(x, comb_r)


def _tc_combine_kernel(x_ref, agg_ref, deg_ref, w_ref, o_ref):
    deg = jnp.sum(deg_ref[...], axis=1)
    den = jnp.maximum(deg, 1.0)
    agg = agg_ref[0] + agg_ref[1]
    nb = agg / den[:, None]
    w = w_ref[...]
    out = lax.dot_general(x_ref[...], w[:, :D], (((1,), (1,)), ((), ())),
                          preferred_element_type=jnp.float32)
    out = out + lax.dot_general(nb, w[:, D:], (((1,), (1,)), ((), ())),
                                preferred_element_type=jnp.float32)
    o_ref[...] = out


def _tc_combine(x, agg_p, deg_p, w):
    blk = 1000
    grid = (N_NODES // blk,)
    return pl.pallas_call(
        _tc_combine_kernel,
        grid=grid,
        in_specs=[
            pl.BlockSpec((blk, D), lambda i: (i, 0)),
            pl.BlockSpec((2, blk, D), lambda i: (0, i, 0)),
            pl.BlockSpec((blk, NW), lambda i: (i, 0)),
            pl.BlockSpec((D, 2 * D), lambda i: (0, 0)),
        ],
        out_specs=pl.BlockSpec((blk, D), lambda i: (i, 0)),
        out_shape=jax.ShapeDtypeStruct((N_NODES, D), jnp.float32),
    )(x, agg_p, deg_p, w)


@jax.jit
def kernel(X, adj, W):
    src = adj[0].astype(jnp.int32)
    dst = adj[1].astype(jnp.int32)
    comb = (src + (dst << 16)).reshape(NW, EPT)
    pad = jnp.full((NW, NCHUNK_P * CHUNK - EPT), PAD_DST << 16, jnp.int32)
    comb = jnp.concatenate([comb, pad], axis=1).reshape(-1)
    agg_p, deg_p = _sc_agg(X, comb)
    deg_p = deg_p.reshape(NW, DEG_N)[:, :N_NODES].T
    return _tc_combine(X, agg_p, deg_p, W)


# speedup vs baseline: 1.0609x; 1.0609x over previous
import functools

import jax
import jax.numpy as jnp
from jax import lax
from jax.experimental import pallas as pl
from jax.experimental.pallas import tpu as pltpu
from jax.experimental.pallas import tpu_sc as plsc

N_NODES = 10000
N_EDGES = 320000
D = 128

NW = 32
CHUNK = 96
NCHUNK = 105
EPT = 10000
N_PAD = 10112
ROWS_PER_TILE = N_PAD // 16
DEG_N = 10008
ZBUF = 160
PAD_DST = 10000


def _sc_agg_kernel(x_hbm, comb_hbm,
                   agg_out, deg_out,
                   comb_v, src_a, dst_a, doff_a, src_b, dst_b,
                   doff_b, rows_a, rows_b, ones_v, zbuf,
                   agg_s, deg_s, sem_a, sem_b):
    c = lax.axis_index("c")
    s = lax.axis_index("s")
    wid = c * 16 + s
    soff = s * DEG_N

    pltpu.sync_copy(comb_hbm.at[wid], comb_v)

    def unpack(j, src_c, dst_c, doff_c):
        for t in range(CHUNK // 16):
            cv = comb_v[j, pl.ds(t * 16, 16)]
            sv = jnp.bitwise_and(cv, 0xFFFF)
            dv = lax.shift_right_logical(cv, 16)
            src_c[pl.ds(t * 16, 16)] = sv
            dst_c[pl.ds(t * 16, 16)] = dv
            doff_c[pl.ds(t * 16, 16)] = dv + soff

    def gather(src_c, rows_v, sem):
        pltpu.async_copy(x_hbm.at[src_c], rows_v, sem)

    def drain(src_c, rows_v, sem):
        pltpu.make_async_copy(x_hbm.at[src_c], rows_v, sem).wait()

    def scatter_rows(dst_c, rows_v):
        pltpu.sync_copy(rows_v, agg_s.at[dst_c], add=True)

    def deg_add(doff_c):
        pltpu.sync_copy(ones_v, deg_s.at[doff_c], add=True)

    ones = jnp.full((16,), 1.0, jnp.float32)
    zeros = jnp.zeros((16,), jnp.float32)
    for t in range(CHUNK // 16):
        ones_v[pl.ds(t * 16, 16)] = ones

    def zrows_body(i, carry):
        r = i // (D // 16)
        t = i % (D // 16)
        rows_b[r, pl.ds(t * 16, 16)] = zeros
        return carry
    lax.fori_loop(0, CHUNK * (D // 16), zrows_body, 0)
    base = s * ROWS_PER_TILE
    for b in range(ROWS_PER_TILE // CHUNK):
        pltpu.sync_copy(rows_b, agg_s.at[pl.ds(base + b * CHUNK, CHUNK)])
    rem = ROWS_PER_TILE % CHUNK
    if rem:
        pltpu.sync_copy(
            rows_b.at[pl.ds(0, rem)],
            agg_s.at[pl.ds(base + (ROWS_PER_TILE // CHUNK) * CHUNK, rem)])

    def zbuf_body(t, carry):
        zbuf[pl.ds(t * 16, 16)] = zeros
        return carry
    lax.fori_loop(0, ZBUF // 16, zbuf_body, 0)
    for i in range(DEG_N // ZBUF):
        pltpu.sync_copy(zbuf, deg_s.at[pl.ds(soff + i * ZBUF, ZBUF)])
    drem = DEG_N % ZBUF
    if drem:
        pltpu.sync_copy(zbuf.at[pl.ds(0, drem)],
                        deg_s.at[pl.ds(soff + (DEG_N // ZBUF) * ZBUF, drem)])

    unpack(0, src_a, dst_a, doff_a)
    gather(src_a, rows_a, sem_a)
    deg_add(doff_a)
    plsc.subcore_barrier()

    def pair_body(k, carry):
        unpack(2 * k + 1, src_b, dst_b, doff_b)
        gather(src_b, rows_b, sem_b)
        drain(src_a, rows_a, sem_a)
        scatter_rows(dst_a, rows_a)
        unpack(2 * k + 2, src_a, dst_a, doff_a)
        gather(src_a, rows_a, sem_a)
        deg_add(doff_a)
        drain(src_b, rows_b, sem_b)
        scatter_rows(dst_b, rows_b)
        deg_add(doff_b)
        return carry

    lax.fori_loop(0, (NCHUNK - 1) // 2, pair_body, 0)
    drain(src_a, rows_a, sem_a)
    scatter_rows(dst_a, rows_a)
    plsc.subcore_barrier()

    pltpu.sync_copy(agg_s.at[pl.ds(base, ROWS_PER_TILE)],
                    agg_out.at[c, pl.ds(base, ROWS_PER_TILE)])
    for i in range(DEG_N // ZBUF):
        pltpu.sync_copy(deg_s.at[pl.ds(soff + i * ZBUF, ZBUF)], zbuf)
        pltpu.sync_copy(zbuf, deg_out.at[pl.ds(wid * DEG_N + i * ZBUF, ZBUF)])
    if DEG_N % ZBUF:
        i = DEG_N // ZBUF
        drem = DEG_N % ZBUF
        pltpu.sync_copy(deg_s.at[pl.ds(soff + i * ZBUF, drem)],
                        zbuf.at[pl.ds(0, drem)])
        pltpu.sync_copy(zbuf.at[pl.ds(0, drem)],
                        deg_out.at[pl.ds(wid * DEG_N + i * ZBUF, drem)])


def _sc_agg(x, comb_r):
    mesh = plsc.VectorSubcoreMesh(core_axis_name="c", subcore_axis_name="s")
    fn = functools.partial(
        pl.kernel,
        mesh=mesh,
        out_type=[
            jax.ShapeDtypeStruct((2, N_PAD, D), jnp.float32),
            jax.ShapeDtypeStruct((NW * DEG_N,), jnp.float32),
        ],
        scratch_types=[
            pltpu.VMEM((NCHUNK, CHUNK), jnp.int32),
            pltpu.VMEM((CHUNK,), jnp.int32),
            pltpu.VMEM((CHUNK,), jnp.int32),
            pltpu.VMEM((CHUNK,), jnp.int32),
            pltpu.VMEM((CHUNK,), jnp.int32),
            pltpu.VMEM((CHUNK,), jnp.int32),
            pltpu.VMEM((CHUNK,), jnp.int32),
            pltpu.VMEM((CHUNK, D), jnp.float32),
            pltpu.VMEM((CHUNK, D), jnp.float32),
            pltpu.VMEM((CHUNK,), jnp.float32),
            pltpu.VMEM((ZBUF,), jnp.float32),
            pltpu.VMEM_SHARED((N_PAD, D), jnp.float32),
            pltpu.VMEM_SHARED((16 * DEG_N,), jnp.float32),
            pltpu.SemaphoreType.DMA,
            pltpu.SemaphoreType.DMA,
        ],
    )
    return fn(_sc_agg_kernel)(x, comb_r)


def _tc_combine_kernel(x_ref, agg_ref, deg_ref, w_ref, o_ref):
    deg = jnp.sum(deg_ref[...], axis=1)
    den = jnp.maximum(deg, 1.0)
    agg = agg_ref[0] + agg_ref[1]
    nb = agg / den[:, None]
    w = w_ref[...]
    out = lax.dot_general(x_ref[...], w[:, :D], (((1,), (1,)), ((), ())),
                          preferred_element_type=jnp.float32)
    out = out + lax.dot_general(nb, w[:, D:], (((1,), (1,)), ((), ())),
                                preferred_element_type=jnp.float32)
    o_ref[...] = out


def _tc_combine(x, agg_p, deg_p, w):
    blk = 1000
    grid = (N_NODES // blk,)
    return pl.pallas_call(
        _tc_combine_kernel,
        grid=grid,
        in_specs=[
            pl.BlockSpec((blk, D), lambda i: (i, 0)),
            pl.BlockSpec((2, blk, D), lambda i: (0, i, 0)),
            pl.BlockSpec((blk, NW), lambda i: (i, 0)),
            pl.BlockSpec((D, 2 * D), lambda i: (0, 0)),
        ],
        out_specs=pl.BlockSpec((blk, D), lambda i: (i, 0)),
        out_shape=jax.ShapeDtypeStruct((N_NODES, D), jnp.float32),
    )(x, agg_p, deg_p, w)


@jax.jit
def kernel(X, adj, W):
    src = adj[0].astype(jnp.int32)
    dst = adj[1].astype(jnp.int32)
    comb = (src + (dst << 16)).reshape(NW, EPT)
    pad = jnp.full((NW, NCHUNK * CHUNK - EPT), PAD_DST << 16, jnp.int32)
    comb = jnp.concatenate([comb, pad], axis=1).reshape(NW, NCHUNK, CHUNK)
    agg_p, deg_p = _sc_agg(X, comb)
    deg_p = deg_p.reshape(NW, DEG_N)[:, :N_NODES].T
    return _tc_combine(X, agg_p, deg_p, W)


# speedup vs baseline: 1.6476x; 1.5530x over previous
import functools

import jax
import jax.numpy as jnp
from jax import lax
from jax.experimental import pallas as pl
from jax.experimental.pallas import tpu as pltpu
from jax.experimental.pallas import tpu_sc as plsc

N_NODES = 10000
N_EDGES = 320000
D = 128

NW = 32
CHUNK = 80
NCHUNK = 125
EPT = 10000
N_PAD = 10112
ROWS_PER_TILE = N_PAD // 16
DEG_N = 10000
ZBUF = 640


def _sc_agg_kernel(x_hbm, comb_hbm,
                   agg_out, deg_out,
                   comb_v, src_a, dst_a, doff_a, src_b, dst_b,
                   doff_b, rows_a, rows_b, ones_v, zbuf,
                   agg_s, deg_s, sem_a, sem_b):
    c = lax.axis_index("c")
    s = lax.axis_index("s")
    wid = c * 16 + s
    soff = s * DEG_N

    pltpu.sync_copy(comb_hbm.at[wid], comb_v)

    def unpack(j, src_c, dst_c, doff_c):
        for t in range(CHUNK // 16):
            cv = comb_v[j, pl.ds(t * 16, 16)]
            sv = jnp.bitwise_and(cv, 0xFFFF)
            dv = lax.shift_right_logical(cv, 16)
            src_c[pl.ds(t * 16, 16)] = sv
            dst_c[pl.ds(t * 16, 16)] = dv
            doff_c[pl.ds(t * 16, 16)] = dv + soff

    def gather(src_c, rows_v, sem):
        pltpu.async_copy(x_hbm.at[src_c], rows_v, sem)

    def drain(src_c, rows_v, sem):
        pltpu.make_async_copy(x_hbm.at[src_c], rows_v, sem).wait()

    def scatter_rows(dst_c, rows_v):
        pltpu.sync_copy(rows_v, agg_s.at[dst_c], add=True)

    def deg_add(doff_c):
        pltpu.sync_copy(ones_v, deg_s.at[doff_c], add=True)

    ones = jnp.full((16,), 1.0, jnp.float32)
    zeros = jnp.zeros((16,), jnp.float32)
    for t in range(CHUNK // 16):
        ones_v[pl.ds(t * 16, 16)] = ones

    def zrows_body(i, carry):
        r = i // (D // 16)
        t = i % (D // 16)
        rows_b[r, pl.ds(t * 16, 16)] = zeros
        return carry
    lax.fori_loop(0, CHUNK * (D // 16), zrows_body, 0)
    base = s * ROWS_PER_TILE
    for b in range(ROWS_PER_TILE // CHUNK):
        pltpu.sync_copy(rows_b, agg_s.at[pl.ds(base + b * CHUNK, CHUNK)])
    rem = ROWS_PER_TILE % CHUNK
    if rem:
        pltpu.sync_copy(
            rows_b.at[pl.ds(0, rem)],
            agg_s.at[pl.ds(base + (ROWS_PER_TILE // CHUNK) * CHUNK, rem)])

    def zbuf_body(t, carry):
        zbuf[pl.ds(t * 16, 16)] = zeros
        return carry
    lax.fori_loop(0, ZBUF // 16, zbuf_body, 0)
    for i in range(DEG_N // ZBUF):
        pltpu.sync_copy(zbuf, deg_s.at[pl.ds(soff + i * ZBUF, ZBUF)])
    drem = DEG_N % ZBUF
    if drem:
        pltpu.sync_copy(zbuf.at[pl.ds(0, drem)],
                        deg_s.at[pl.ds(soff + (DEG_N // ZBUF) * ZBUF, drem)])

    unpack(0, src_a, dst_a, doff_a)
    gather(src_a, rows_a, sem_a)
    deg_add(doff_a)
    plsc.subcore_barrier()

    def pair_body(k, carry):
        unpack(2 * k + 1, src_b, dst_b, doff_b)
        gather(src_b, rows_b, sem_b)
        drain(src_a, rows_a, sem_a)
        scatter_rows(dst_a, rows_a)
        unpack(2 * k + 2, src_a, dst_a, doff_a)
        gather(src_a, rows_a, sem_a)
        deg_add(doff_a)
        drain(src_b, rows_b, sem_b)
        scatter_rows(dst_b, rows_b)
        deg_add(doff_b)
        return carry

    lax.fori_loop(0, (NCHUNK - 1) // 2, pair_body, 0)
    drain(src_a, rows_a, sem_a)
    scatter_rows(dst_a, rows_a)
    plsc.subcore_barrier()

    pltpu.sync_copy(agg_s.at[pl.ds(base, ROWS_PER_TILE)],
                    agg_out.at[c, pl.ds(base, ROWS_PER_TILE)])
    for i in range(DEG_N // ZBUF):
        pltpu.sync_copy(deg_s.at[pl.ds(soff + i * ZBUF, ZBUF)], zbuf)
        pltpu.sync_copy(zbuf, deg_out.at[pl.ds(wid * DEG_N + i * ZBUF, ZBUF)])
    if DEG_N % ZBUF:
        i = DEG_N // ZBUF
        drem = DEG_N % ZBUF
        pltpu.sync_copy(deg_s.at[pl.ds(soff + i * ZBUF, drem)],
                        zbuf.at[pl.ds(0, drem)])
        pltpu.sync_copy(zbuf.at[pl.ds(0, drem)],
                        deg_out.at[pl.ds(wid * DEG_N + i * ZBUF, drem)])


def _sc_agg(x, comb_r):
    mesh = plsc.VectorSubcoreMesh(core_axis_name="c", subcore_axis_name="s")
    fn = functools.partial(
        pl.kernel,
        mesh=mesh,
        out_type=[
            jax.ShapeDtypeStruct((2, N_PAD, D), jnp.float32),
            jax.ShapeDtypeStruct((NW * DEG_N,), jnp.float32),
        ],
        scratch_types=[
            pltpu.VMEM((NCHUNK, CHUNK), jnp.int32),
            pltpu.VMEM((CHUNK,), jnp.int32),
            pltpu.VMEM((CHUNK,), jnp.int32),
            pltpu.VMEM((CHUNK,), jnp.int32),
            pltpu.VMEM((CHUNK,), jnp.int32),
            pltpu.VMEM((CHUNK,), jnp.int32),
            pltpu.VMEM((CHUNK,), jnp.int32),
            pltpu.VMEM((CHUNK, D), jnp.float32),
            pltpu.VMEM((CHUNK, D), jnp.float32),
            pltpu.VMEM((CHUNK,), jnp.float32),
            pltpu.VMEM((ZBUF,), jnp.float32),
            pltpu.VMEM_SHARED((N_PAD, D), jnp.float32),
            pltpu.VMEM_SHARED((16 * DEG_N,), jnp.float32),
            pltpu.SemaphoreType.DMA,
            pltpu.SemaphoreType.DMA,
        ],
    )
    return fn(_sc_agg_kernel)(x, comb_r)


def _tc_combine_kernel(x_ref, agg_ref, deg_ref, w_ref, o_ref):
    deg = jnp.sum(deg_ref[...], axis=1)
    den = jnp.maximum(deg, 1.0)
    agg = agg_ref[0] + agg_ref[1]
    nb = agg / den[:, None]
    w = w_ref[...]
    out = lax.dot_general(x_ref[...], w[:, :D], (((1,), (1,)), ((), ())),
                          preferred_element_type=jnp.float32)
    out = out + lax.dot_general(nb, w[:, D:], (((1,), (1,)), ((), ())),
                                preferred_element_type=jnp.float32)
    o_ref[...] = out


def _tc_combine(x, agg_p, deg_p, w):
    blk = 1000
    grid = (N_NODES // blk,)
    return pl.pallas_call(
        _tc_combine_kernel,
        grid=grid,
        in_specs=[
            pl.BlockSpec((blk, D), lambda i: (i, 0)),
            pl.BlockSpec((2, blk, D), lambda i: (0, i, 0)),
            pl.BlockSpec((blk, NW), lambda i: (i, 0)),
            pl.BlockSpec((D, 2 * D), lambda i: (0, 0)),
        ],
        out_specs=pl.BlockSpec((blk, D), lambda i: (i, 0)),
        out_shape=jax.ShapeDtypeStruct((N_NODES, D), jnp.float32),
    )(x, agg_p, deg_p, w)


@jax.jit
def kernel(X, adj, W):
    src = adj[0].astype(jnp.int32)
    dst = adj[1].astype(jnp.int32)
    comb = (src + (dst << 16)).reshape(NW, NCHUNK, CHUNK)
    agg_p, deg_p = _sc_agg(X, comb)
    deg_p = deg_p.reshape(NW, DEG_N)[:, :N_NODES].T
    return _tc_combine(X, agg_p, deg_p, W)


# speedup vs baseline: 1.6481x; 1.0003x over previous
import functools

import jax
import jax.numpy as jnp
from jax import lax
from jax.experimental import pallas as pl
from jax.experimental.pallas import tpu as pltpu
from jax.experimental.pallas import tpu_sc as plsc

N_NODES = 10000
N_EDGES = 320000
D = 128

NW = 32
CHUNK = 80
NCHUNK = 125
EPT = 10000
N_PAD = 10112
ROWS_PER_TILE = N_PAD // 16
DEG_N = 10000
ZBUF = 640


def _sc_agg_kernel(x_hbm, comb_hbm,
                   agg_out, deg_out,
                   comb_v, src_a, dst_a, doff_a, src_b, dst_b,
                   doff_b, rows_a, rows_b, ones_v, zbuf,
                   agg_s, deg_s, sem_a, sem_b):
    c = lax.axis_index("c")
    s = lax.axis_index("s")
    wid = c * 16 + s
    soff = s * DEG_N

    pltpu.sync_copy(comb_hbm.at[wid], comb_v)

    def unpack(j, src_c, dst_c, doff_c):
        for t in range(CHUNK // 16):
            cv = comb_v[j, pl.ds(t * 16, 16)]
            sv = jnp.bitwise_and(cv, 0xFFFF)
            dv = lax.shift_right_logical(cv, 16)
            src_c[pl.ds(t * 16, 16)] = sv
            dst_c[pl.ds(t * 16, 16)] = dv
            doff_c[pl.ds(t * 16, 16)] = dv + soff

    def gather(src_c, rows_v, sem):
        pltpu.async_copy(x_hbm.at[src_c], rows_v, sem)

    def drain(src_c, rows_v, sem):
        pltpu.make_async_copy(x_hbm.at[src_c], rows_v, sem).wait()

    def scatter_rows(dst_c, rows_v):
        pltpu.sync_copy(rows_v, agg_s.at[dst_c], add=True)

    def deg_add(doff_c):
        pltpu.sync_copy(ones_v, deg_s.at[doff_c], add=True)

    ones = jnp.full((16,), 1.0, jnp.float32)
    zeros = jnp.zeros((16,), jnp.float32)
    for t in range(CHUNK // 16):
        ones_v[pl.ds(t * 16, 16)] = ones

    def zrows_body(i, carry):
        r = i // (D // 16)
        t = i % (D // 16)
        rows_b[r, pl.ds(t * 16, 16)] = zeros
        return carry
    lax.fori_loop(0, CHUNK * (D // 16), zrows_body, 0)
    base = s * ROWS_PER_TILE
    for b in range(ROWS_PER_TILE // CHUNK):
        pltpu.sync_copy(rows_b, agg_s.at[pl.ds(base + b * CHUNK, CHUNK)])
    rem = ROWS_PER_TILE % CHUNK
    if rem:
        pltpu.sync_copy(
            rows_b.at[pl.ds(0, rem)],
            agg_s.at[pl.ds(base + (ROWS_PER_TILE // CHUNK) * CHUNK, rem)])

    def zbuf_body(t, carry):
        zbuf[pl.ds(t * 16, 16)] = zeros
        return carry
    lax.fori_loop(0, ZBUF // 16, zbuf_body, 0)
    for i in range(DEG_N // ZBUF):
        pltpu.sync_copy(zbuf, deg_s.at[pl.ds(soff + i * ZBUF, ZBUF)])
    drem = DEG_N % ZBUF
    if drem:
        pltpu.sync_copy(zbuf.at[pl.ds(0, drem)],
                        deg_s.at[pl.ds(soff + (DEG_N // ZBUF) * ZBUF, drem)])

    unpack(0, src_a, dst_a, doff_a)
    gather(src_a, rows_a, sem_a)
    deg_add(doff_a)
    plsc.subcore_barrier()

    def pair_body(k, carry):
        unpack(2 * k + 1, src_b, dst_b, doff_b)
        gather(src_b, rows_b, sem_b)
        drain(src_a, rows_a, sem_a)
        scatter_rows(dst_a, rows_a)
        unpack(2 * k + 2, src_a, dst_a, doff_a)
        gather(src_a, rows_a, sem_a)
        deg_add(doff_a)
        drain(src_b, rows_b, sem_b)
        scatter_rows(dst_b, rows_b)
        deg_add(doff_b)
        return carry

    lax.fori_loop(0, (NCHUNK - 1) // 2, pair_body, 0)
    drain(src_a, rows_a, sem_a)
    scatter_rows(dst_a, rows_a)
    plsc.subcore_barrier()

    pltpu.sync_copy(agg_s.at[pl.ds(base, ROWS_PER_TILE)],
                    agg_out.at[c, pl.ds(base, ROWS_PER_TILE)])
    for i in range(DEG_N // ZBUF):
        pltpu.sync_copy(deg_s.at[pl.ds(soff + i * ZBUF, ZBUF)], zbuf)
        pltpu.sync_copy(zbuf, deg_out.at[pl.ds(wid * DEG_N + i * ZBUF, ZBUF)])
    if DEG_N % ZBUF:
        i = DEG_N // ZBUF
        drem = DEG_N % ZBUF
        pltpu.sync_copy(deg_s.at[pl.ds(soff + i * ZBUF, drem)],
                        zbuf.at[pl.ds(0, drem)])
        pltpu.sync_copy(zbuf.at[pl.ds(0, drem)],
                        deg_out.at[pl.ds(wid * DEG_N + i * ZBUF, drem)])


def _sc_agg(x, comb_r):
    mesh = plsc.VectorSubcoreMesh(core_axis_name="c", subcore_axis_name="s")
    fn = functools.partial(
        pl.kernel,
        mesh=mesh,
        out_type=[
            jax.ShapeDtypeStruct((2, N_PAD, D), jnp.float32),
            jax.ShapeDtypeStruct((NW * DEG_N,), jnp.float32),
        ],
        scratch_types=[
            pltpu.VMEM((NCHUNK, CHUNK), jnp.int32),
            pltpu.VMEM((CHUNK,), jnp.int32),
            pltpu.VMEM((CHUNK,), jnp.int32),
            pltpu.VMEM((CHUNK,), jnp.int32),
            pltpu.VMEM((CHUNK,), jnp.int32),
            pltpu.VMEM((CHUNK,), jnp.int32),
            pltpu.VMEM((CHUNK,), jnp.int32),
            pltpu.VMEM((CHUNK, D), jnp.float32),
            pltpu.VMEM((CHUNK, D), jnp.float32),
            pltpu.VMEM((CHUNK,), jnp.float32),
            pltpu.VMEM((ZBUF,), jnp.float32),
            pltpu.VMEM_SHARED((N_PAD, D), jnp.float32),
            pltpu.VMEM_SHARED((16 * DEG_N,), jnp.float32),
            pltpu.SemaphoreType.DMA,
            pltpu.SemaphoreType.DMA,
        ],
    )
    return fn(_sc_agg_kernel)(x, comb_r)


def _tc_matmul1_kernel(x_ref, w_ref, o_ref):
    o_ref[...] = lax.dot_general(
        x_ref[...], w_ref[...][:, :D], (((1,), (1,)), ((), ())),
        preferred_element_type=jnp.float32)


def _tc_matmul1(x, w):
    blk = 1000
    return pl.pallas_call(
        _tc_matmul1_kernel,
        grid=(N_NODES // blk,),
        in_specs=[
            pl.BlockSpec((blk, D), lambda i: (i, 0)),
            pl.BlockSpec((D, 2 * D), lambda i: (0, 0)),
        ],
        out_specs=pl.BlockSpec((blk, D), lambda i: (i, 0)),
        out_shape=jax.ShapeDtypeStruct((N_NODES, D), jnp.float32),
    )(x, w)


def _tc_combine_kernel(out1_ref, agg_ref, deg_ref, w_ref, o_ref):
    deg = jnp.sum(deg_ref[...], axis=1)
    den = jnp.maximum(deg, 1.0)
    agg = agg_ref[0] + agg_ref[1]
    nb = agg / den[:, None]
    out = out1_ref[...] + lax.dot_general(
        nb, w_ref[...][:, D:], (((1,), (1,)), ((), ())),
        preferred_element_type=jnp.float32)
    o_ref[...] = out


def _tc_combine(out1, agg_p, deg_p, w):
    blk = 1000
    grid = (N_NODES // blk,)
    return pl.pallas_call(
        _tc_combine_kernel,
        grid=grid,
        in_specs=[
            pl.BlockSpec((blk, D), lambda i: (i, 0)),
            pl.BlockSpec((2, blk, D), lambda i: (0, i, 0)),
            pl.BlockSpec((blk, NW), lambda i: (i, 0)),
            pl.BlockSpec((D, 2 * D), lambda i: (0, 0)),
        ],
        out_specs=pl.BlockSpec((blk, D), lambda i: (i, 0)),
        out_shape=jax.ShapeDtypeStruct((N_NODES, D), jnp.float32),
    )(out1, agg_p, deg_p, w)


@jax.jit
def kernel(X, adj, W):
    src = adj[0].astype(jnp.int32)
    dst = adj[1].astype(jnp.int32)
    comb = (src + (dst << 16)).reshape(NW, NCHUNK, CHUNK)
    agg_p, deg_p = _sc_agg(X, comb)
    out1 = _tc_matmul1(X, W)
    deg_p = deg_p.reshape(NW, DEG_N)[:, :N_NODES].T
    return _tc_combine(out1, agg_p, deg_p, W)


# speedup vs baseline: 1.6939x; 1.0278x over previous
import functools

import jax
import jax.numpy as jnp
from jax import lax
from jax.experimental import pallas as pl
from jax.experimental.pallas import tpu as pltpu
from jax.experimental.pallas import tpu_sc as plsc

N_NODES = 10000
N_EDGES = 320000
D = 128

NW = 32
CHUNK = 80
NCHUNK = 125
EPT = 10000
N_PAD = 10112
ROWS_PER_TILE = N_PAD // 16
DEG_N = 10000
ZBUF = 640


def _sc_agg_kernel(x_hbm, comb_hbm,
                   agg_out, deg_out,
                   comb_v, src_a, dst_a, doff_a, src_b, dst_b,
                   doff_b, rows_a, rows_b, ones_v, zbuf,
                   agg_s, deg_s, sem_a, sem_b):
    c = lax.axis_index("c")
    s = lax.axis_index("s")
    wid = c * 16 + s
    soff = s * DEG_N

    pltpu.sync_copy(comb_hbm.at[wid], comb_v)

    def unpack(j, src_c, dst_c, doff_c):
        for t in range(CHUNK // 16):
            cv = comb_v[j, pl.ds(t * 16, 16)]
            sv = jnp.bitwise_and(cv, 0xFFFF)
            dv = lax.shift_right_logical(cv, 16)
            src_c[pl.ds(t * 16, 16)] = sv
            dst_c[pl.ds(t * 16, 16)] = dv
            doff_c[pl.ds(t * 16, 16)] = dv + soff

    def gather(src_c, rows_v, sem):
        pltpu.async_copy(x_hbm.at[src_c], rows_v, sem)

    def drain(src_c, rows_v, sem):
        pltpu.make_async_copy(x_hbm.at[src_c], rows_v, sem).wait()

    def scatter_rows(dst_c, rows_v):
        pltpu.sync_copy(rows_v, agg_s.at[dst_c], add=True)

    def deg_add(doff_c):
        pltpu.sync_copy(ones_v, deg_s.at[doff_c], add=True)

    ones = jnp.full((16,), 1.0, jnp.float32)
    zeros = jnp.zeros((16,), jnp.float32)
    for t in range(CHUNK // 16):
        ones_v[pl.ds(t * 16, 16)] = ones

    def zrows_body(i, carry):
        r = i // (D // 16)
        t = i % (D // 16)
        rows_b[r, pl.ds(t * 16, 16)] = zeros
        return carry
    lax.fori_loop(0, CHUNK * (D // 16), zrows_body, 0)
    base = s * ROWS_PER_TILE
    for b in range(ROWS_PER_TILE // CHUNK):
        pltpu.sync_copy(rows_b, agg_s.at[pl.ds(base + b * CHUNK, CHUNK)])
    rem = ROWS_PER_TILE % CHUNK
    if rem:
        pltpu.sync_copy(
            rows_b.at[pl.ds(0, rem)],
            agg_s.at[pl.ds(base + (ROWS_PER_TILE // CHUNK) * CHUNK, rem)])

    def zbuf_body(t, carry):
        zbuf[pl.ds(t * 16, 16)] = zeros
        return carry
    lax.fori_loop(0, ZBUF // 16, zbuf_body, 0)
    for i in range(DEG_N // ZBUF):
        pltpu.sync_copy(zbuf, deg_s.at[pl.ds(soff + i * ZBUF, ZBUF)])
    drem = DEG_N % ZBUF
    if drem:
        pltpu.sync_copy(zbuf.at[pl.ds(0, drem)],
                        deg_s.at[pl.ds(soff + (DEG_N // ZBUF) * ZBUF, drem)])

    unpack(0, src_a, dst_a, doff_a)
    gather(src_a, rows_a, sem_a)
    deg_add(doff_a)
    plsc.subcore_barrier()

    def pair_body(k, carry):
        unpack(2 * k + 1, src_b, dst_b, doff_b)
        gather(src_b, rows_b, sem_b)
        deg_add(doff_b)
        drain(src_a, rows_a, sem_a)
        scatter_rows(dst_a, rows_a)
        unpack(2 * k + 2, src_a, dst_a, doff_a)
        gather(src_a, rows_a, sem_a)
        deg_add(doff_a)
        drain(src_b, rows_b, sem_b)
        scatter_rows(dst_b, rows_b)
        return carry

    lax.fori_loop(0, (NCHUNK - 1) // 2, pair_body, 0)
    drain(src_a, rows_a, sem_a)
    scatter_rows(dst_a, rows_a)
    plsc.subcore_barrier()

    pltpu.sync_copy(agg_s.at[pl.ds(base, ROWS_PER_TILE)],
                    agg_out.at[c, pl.ds(base, ROWS_PER_TILE)])
    for i in range(DEG_N // ZBUF):
        pltpu.sync_copy(deg_s.at[pl.ds(soff + i * ZBUF, ZBUF)], zbuf)
        pltpu.sync_copy(zbuf, deg_out.at[pl.ds(wid * DEG_N + i * ZBUF, ZBUF)])
    if DEG_N % ZBUF:
        i = DEG_N // ZBUF
        drem = DEG_N % ZBUF
        pltpu.sync_copy(deg_s.at[pl.ds(soff + i * ZBUF, drem)],
                        zbuf.at[pl.ds(0, drem)])
        pltpu.sync_copy(zbuf.at[pl.ds(0, drem)],
                        deg_out.at[pl.ds(wid * DEG_N + i * ZBUF, drem)])


def _sc_agg(x, comb_r):
    mesh = plsc.VectorSubcoreMesh(core_axis_name="c", subcore_axis_name="s")
    fn = functools.partial(
        pl.kernel,
        mesh=mesh,
        out_type=[
            jax.ShapeDtypeStruct((2, N_PAD, D), jnp.float32),
            jax.ShapeDtypeStruct((NW * DEG_N,), jnp.float32),
        ],
        scratch_types=[
            pltpu.VMEM((NCHUNK, CHUNK), jnp.int32),
            pltpu.VMEM((CHUNK,), jnp.int32),
            pltpu.VMEM((CHUNK,), jnp.int32),
            pltpu.VMEM((CHUNK,), jnp.int32),
            pltpu.VMEM((CHUNK,), jnp.int32),
            pltpu.VMEM((CHUNK,), jnp.int32),
            pltpu.VMEM((CHUNK,), jnp.int32),
            pltpu.VMEM((CHUNK, D), jnp.float32),
            pltpu.VMEM((CHUNK, D), jnp.float32),
            pltpu.VMEM((CHUNK,), jnp.float32),
            pltpu.VMEM((ZBUF,), jnp.float32),
            pltpu.VMEM_SHARED((N_PAD, D), jnp.float32),
            pltpu.VMEM_SHARED((16 * DEG_N,), jnp.float32),
            pltpu.SemaphoreType.DMA,
            pltpu.SemaphoreType.DMA,
        ],
    )
    return fn(_sc_agg_kernel)(x, comb_r)


def _tc_matmul1_kernel(x_ref, w_ref, o_ref):
    o_ref[...] = lax.dot_general(
        x_ref[...], w_ref[...][:, :D], (((1,), (1,)), ((), ())),
        preferred_element_type=jnp.float32)


def _tc_matmul1(x, w):
    blk = 1000
    return pl.pallas_call(
        _tc_matmul1_kernel,
        grid=(N_NODES // blk,),
        in_specs=[
            pl.BlockSpec((blk, D), lambda i: (i, 0)),
            pl.BlockSpec((D, 2 * D), lambda i: (0, 0)),
        ],
        out_specs=pl.BlockSpec((blk, D), lambda i: (i, 0)),
        out_shape=jax.ShapeDtypeStruct((N_NODES, D), jnp.float32),
    )(x, w)


def _tc_combine_kernel(out1_ref, agg_ref, deg_ref, w_ref, o_ref):
    deg = jnp.sum(deg_ref[...], axis=1)
    den = jnp.maximum(deg, 1.0)
    agg = agg_ref[0] + agg_ref[1]
    nb = agg / den[:, None]
    out = out1_ref[...] + lax.dot_general(
        nb, w_ref[...][:, D:], (((1,), (1,)), ((), ())),
        preferred_element_type=jnp.float32)
    o_ref[...] = out


def _tc_combine(out1, agg_p, deg_p, w):
    blk = 1000
    grid = (N_NODES // blk,)
    return pl.pallas_call(
        _tc_combine_kernel,
        grid=grid,
        in_specs=[
            pl.BlockSpec((blk, D), lambda i: (i, 0)),
            pl.BlockSpec((2, blk, D), lambda i: (0, i, 0)),
            pl.BlockSpec((blk, NW), lambda i: (i, 0)),
            pl.BlockSpec((D, 2 * D), lambda i: (0, 0)),
        ],
        out_specs=pl.BlockSpec((blk, D), lambda i: (i, 0)),
        out_shape=jax.ShapeDtypeStruct((N_NODES, D), jnp.float32),
    )(out1, agg_p, deg_p, w)


@jax.jit
def kernel(X, adj, W):
    src = adj[0].astype(jnp.int32)
    dst = adj[1].astype(jnp.int32)
    comb = (src + (dst << 16)).reshape(NW, NCHUNK, CHUNK)
    agg_p, deg_p = _sc_agg(X, comb)
    out1 = _tc_matmul1(X, W)
    deg_p = deg_p.reshape(NW, DEG_N)[:, :N_NODES].T
    return _tc_combine(out1, agg_p, deg_p, W)


# speedup vs baseline: 1.7197x; 1.0152x over previous
import functools

import jax
import jax.numpy as jnp
from jax import lax
from jax.experimental import pallas as pl
from jax.experimental.pallas import tpu as pltpu
from jax.experimental.pallas import tpu_sc as plsc

N_NODES = 10000
N_EDGES = 320000
D = 128

NW = 32
CHUNK = 80
NCHUNK = 125
EPT = 10000
N_PAD = 10112
ROWS_PER_TILE = N_PAD // 16
DEG_N = 10000
ZBUF = 1280


def _sc_agg_kernel(x_hbm, comb_hbm,
                   agg_out, deg_out,
                   comb_v, src_a, dst_a, doff_a, src_b, dst_b,
                   doff_b, rows_a, rows_b, ones_v, zbuf,
                   agg_s, deg_s, sem_a, sem_b):
    c = lax.axis_index("c")
    s = lax.axis_index("s")
    wid = c * 16 + s
    soff = s * DEG_N

    pltpu.async_copy(comb_hbm.at[wid], comb_v, sem_a)

    def unpack(j, src_c, dst_c, doff_c):
        for t in range(CHUNK // 16):
            cv = comb_v[j, pl.ds(t * 16, 16)]
            sv = jnp.bitwise_and(cv, 0xFFFF)
            dv = lax.shift_right_logical(cv, 16)
            src_c[pl.ds(t * 16, 16)] = sv
            dst_c[pl.ds(t * 16, 16)] = dv
            doff_c[pl.ds(t * 16, 16)] = dv + soff

    def gather(src_c, rows_v, sem):
        pltpu.async_copy(x_hbm.at[src_c], rows_v, sem)

    def drain(src_c, rows_v, sem):
        pltpu.make_async_copy(x_hbm.at[src_c], rows_v, sem).wait()

    def scatter_rows(dst_c, rows_v):
        pltpu.sync_copy(rows_v, agg_s.at[dst_c], add=True)

    def deg_add(doff_c):
        pltpu.sync_copy(ones_v, deg_s.at[doff_c], add=True)

    ones = jnp.full((16,), 1.0, jnp.float32)
    zeros = jnp.zeros((16,), jnp.float32)
    for t in range(CHUNK // 16):
        ones_v[pl.ds(t * 16, 16)] = ones

    def zrows_body(i, carry):
        r = i // (D // 16)
        t = i % (D // 16)
        rows_b[r, pl.ds(t * 16, 16)] = zeros
        return carry
    lax.fori_loop(0, CHUNK * (D // 16), zrows_body, 0)
    base = s * ROWS_PER_TILE
    for b in range(ROWS_PER_TILE // CHUNK):
        pltpu.sync_copy(rows_b, agg_s.at[pl.ds(base + b * CHUNK, CHUNK)])
    rem = ROWS_PER_TILE % CHUNK
    if rem:
        pltpu.sync_copy(
            rows_b.at[pl.ds(0, rem)],
            agg_s.at[pl.ds(base + (ROWS_PER_TILE // CHUNK) * CHUNK, rem)])

    def zbuf_body(t, carry):
        zbuf[pl.ds(t * 16, 16)] = zeros
        return carry
    lax.fori_loop(0, ZBUF // 16, zbuf_body, 0)
    for i in range(DEG_N // ZBUF):
        pltpu.sync_copy(zbuf, deg_s.at[pl.ds(soff + i * ZBUF, ZBUF)])
    drem = DEG_N % ZBUF
    if drem:
        pltpu.sync_copy(zbuf.at[pl.ds(0, drem)],
                        deg_s.at[pl.ds(soff + (DEG_N // ZBUF) * ZBUF, drem)])

    pltpu.make_async_copy(comb_hbm.at[wid], comb_v, sem_a).wait()
    unpack(0, src_a, dst_a, doff_a)
    gather(src_a, rows_a, sem_a)
    deg_add(doff_a)
    plsc.subcore_barrier()

    def pair_body(k, carry):
        unpack(2 * k + 1, src_b, dst_b, doff_b)
        gather(src_b, rows_b, sem_b)
        deg_add(doff_b)
        drain(src_a, rows_a, sem_a)
        scatter_rows(dst_a, rows_a)
        unpack(2 * k + 2, src_a, dst_a, doff_a)
        gather(src_a, rows_a, sem_a)
        deg_add(doff_a)
        drain(src_b, rows_b, sem_b)
        scatter_rows(dst_b, rows_b)
        return carry

    lax.fori_loop(0, (NCHUNK - 1) // 2, pair_body, 0)
    drain(src_a, rows_a, sem_a)
    scatter_rows(dst_a, rows_a)
    plsc.subcore_barrier()

    pltpu.sync_copy(agg_s.at[pl.ds(base, ROWS_PER_TILE)],
                    agg_out.at[c, pl.ds(base, ROWS_PER_TILE)])
    for i in range(DEG_N // ZBUF):
        pltpu.sync_copy(deg_s.at[pl.ds(soff + i * ZBUF, ZBUF)], zbuf)
        pltpu.sync_copy(zbuf, deg_out.at[pl.ds(wid * DEG_N + i * ZBUF, ZBUF)])
    if DEG_N % ZBUF:
        i = DEG_N // ZBUF
        drem = DEG_N % ZBUF
        pltpu.sync_copy(deg_s.at[pl.ds(soff + i * ZBUF, drem)],
                        zbuf.at[pl.ds(0, drem)])
        pltpu.sync_copy(zbuf.at[pl.ds(0, drem)],
                        deg_out.at[pl.ds(wid * DEG_N + i * ZBUF, drem)])


def _sc_agg(x, comb_r):
    mesh = plsc.VectorSubcoreMesh(core_axis_name="c", subcore_axis_name="s")
    fn = functools.partial(
        pl.kernel,
        mesh=mesh,
        out_type=[
            jax.ShapeDtypeStruct((2, N_PAD, D), jnp.float32),
            jax.ShapeDtypeStruct((NW * DEG_N,), jnp.float32),
        ],
        scratch_types=[
            pltpu.VMEM((NCHUNK, CHUNK), jnp.int32),
            pltpu.VMEM((CHUNK,), jnp.int32),
            pltpu.VMEM((CHUNK,), jnp.int32),
            pltpu.VMEM((CHUNK,), jnp.int32),
            pltpu.VMEM((CHUNK,), jnp.int32),
            pltpu.VMEM((CHUNK,), jnp.int32),
            pltpu.VMEM((CHUNK,), jnp.int32),
            pltpu.VMEM((CHUNK, D), jnp.float32),
            pltpu.VMEM((CHUNK, D), jnp.float32),
            pltpu.VMEM((CHUNK,), jnp.float32),
            pltpu.VMEM((ZBUF,), jnp.float32),
            pltpu.VMEM_SHARED((N_PAD, D), jnp.float32),
            pltpu.VMEM_SHARED((16 * DEG_N,), jnp.float32),
            pltpu.SemaphoreType.DMA,
            pltpu.SemaphoreType.DMA,
        ],
    )
    return fn(_sc_agg_kernel)(x, comb_r)


def _tc_matmul1_kernel(x_ref, w_ref, o_ref):
    o_ref[...] = lax.dot_general(
        x_ref[...], w_ref[...][:, :D], (((1,), (1,)), ((), ())),
        preferred_element_type=jnp.float32)


def _tc_matmul1(x, w):
    blk = 1000
    return pl.pallas_call(
        _tc_matmul1_kernel,
        grid=(N_NODES // blk,),
        in_specs=[
            pl.BlockSpec((blk, D), lambda i: (i, 0)),
            pl.BlockSpec((D, 2 * D), lambda i: (0, 0)),
        ],
        out_specs=pl.BlockSpec((blk, D), lambda i: (i, 0)),
        out_shape=jax.ShapeDtypeStruct((N_NODES, D), jnp.float32),
    )(x, w)


def _tc_combine_kernel(out1_ref, agg_ref, deg_ref, w_ref, o_ref):
    deg = jnp.sum(deg_ref[...], axis=1)
    den = jnp.maximum(deg, 1.0)
    agg = agg_ref[0] + agg_ref[1]
    nb = agg / den[:, None]
    out = out1_ref[...] + lax.dot_general(
        nb, w_ref[...][:, D:], (((1,), (1,)), ((), ())),
        preferred_element_type=jnp.float32)
    o_ref[...] = out


def _tc_combine(out1, agg_p, deg_p, w):
    blk = 1000
    grid = (N_NODES // blk,)
    return pl.pallas_call(
        _tc_combine_kernel,
        grid=grid,
        in_specs=[
            pl.BlockSpec((blk, D), lambda i: (i, 0)),
            pl.BlockSpec((2, blk, D), lambda i: (0, i, 0)),
            pl.BlockSpec((blk, NW), lambda i: (i, 0)),
            pl.BlockSpec((D, 2 * D), lambda i: (0, 0)),
        ],
        out_specs=pl.BlockSpec((blk, D), lambda i: (i, 0)),
        out_shape=jax.ShapeDtypeStruct((N_NODES, D), jnp.float32),
    )(out1, agg_p, deg_p, w)


@jax.jit
def kernel(X, adj, W):
    src = adj[0].astype(jnp.int32)
    dst = adj[1].astype(jnp.int32)
    comb = (src + (dst << 16)).reshape(NW, NCHUNK, CHUNK)
    agg_p, deg_p = _sc_agg(X, comb)
    out1 = _tc_matmul1(X, W)
    deg_p = deg_p.reshape(NW, DEG_N)[:, :N_NODES].T
    return _tc_combine(out1, agg_p, deg_p, W)
